# baseline (speedup 1.0000x reference)
import functools

import jax
import jax.numpy as jnp
from jax import lax
from jax.experimental import pallas as pl
from jax.experimental.pallas import tpu as pltpu
from jax.experimental.pallas import tpu_sc as plsc

D = 128
NC, NS = 2, 16
NW = NC * NS
C = 40


def _prep_body(x_ref, wm1_ref, bm1_ref, ww1_ref, bw1_ref,
               pam_ref, pbm_ref, paw_ref, pbw_ref):
    x = x_ref[...]
    pam_ref[...] = jnp.dot(x, wm1_ref[0:D, :], preferred_element_type=jnp.float32) + bm1_ref[...]
    pbm_ref[...] = jnp.dot(x, wm1_ref[D:2 * D, :], preferred_element_type=jnp.float32)
    paw_ref[...] = jnp.dot(x, ww1_ref[0:D, :], preferred_element_type=jnp.float32) + bw1_ref[...]
    pbw_ref[...] = jnp.dot(x, ww1_ref[D:2 * D, :], preferred_element_type=jnp.float32)


def _precompute_tables(x, wm1, bm1, ww1, bw1):
    n = x.shape[0]
    blk = n // 5
    tbl = jax.ShapeDtypeStruct((n, D), jnp.float32)
    row_spec = pl.BlockSpec((blk, D), lambda i: (i, 0))
    full = pl.BlockSpec((2 * D, D), lambda i: (0, 0))
    bias = pl.BlockSpec((1, D), lambda i: (0, 0))
    return pl.pallas_call(
        _prep_body,
        grid=(5,),
        in_specs=[row_spec, full, bias, full, bias],
        out_specs=(row_spec, row_spec, row_spec, row_spec),
        out_shape=(tbl, tbl, tbl, tbl),
    )(x, wm1[: 2 * D], bm1.reshape(1, D), ww1[: 2 * D], bw1.reshape(1, D))


def _add_chunk(a, b, w):
    def row(r, carry):
        for j in range(D // 16):
            sl = pl.ds(j * 16, 16)
            w[r, sl] = a[r, sl] + b[r, sl]
        return carry
    lax.fori_loop(0, C, row, 0, unroll=4)


def _fused_gather_add(ta, tb, idxs_v, idxr_v, out, base, n_edges,
                      a0, a1, b0, b1, w0, w1, gsem, wsem):
    n = n_edges // C

    def issue(jj, sa, sb):
        pltpu.async_copy(ta.at[idxs_v.at[pl.ds(jj * C, C)]], sa, gsem)
        pltpu.async_copy(tb.at[idxr_v.at[pl.ds(jj * C, C)]], sb, gsem)

    def wait_gathers(sa, sb):
        pltpu.make_async_copy(ta.at[pl.ds(0, C)], sa, gsem).wait()
        pltpu.make_async_copy(tb.at[pl.ds(0, C)], sb, gsem).wait()

    def wait_write(sw):
        pltpu.make_async_copy(sw, out.at[pl.ds(base, C)], wsem).wait()

    issue(0, a0, b0)

    def pair(k, carry):
        @pl.when(k >= 1)
        def _():
            wait_write(w0)
        wait_gathers(a0, b0)
        issue(2 * k + 1, a1, b1)
        _add_chunk(a0, b0, w0)
        pltpu.async_copy(w0, out.at[pl.ds(base + (2 * k) * C, C)], wsem)
        @pl.when(k >= 1)
        def _():
            wait_write(w1)
        wait_gathers(a1, b1)

        @pl.when(2 * k + 2 < n)
        def _():
            issue(2 * k + 2, a0, b0)
        _add_chunk(a1, b1, w1)
        pltpu.async_copy(w1, out.at[pl.ds(base + (2 * k + 1) * C, C)], wsem)
        return carry

    lax.fori_loop(0, n // 2, pair, 0)
    wait_write(w0)
    wait_write(w1)


def _sc_body(ta, tb, s_idx, r_idx, out,
             idxs_v, idxr_v, a0, a1, b0, b1, w0, w1, gsem, wsem):
    wid = lax.axis_index("s") * NC + lax.axis_index("c")
    e_per = s_idx.shape[0] // NW
    base = wid * e_per
    pltpu.sync_copy(s_idx.at[pl.ds(base, e_per)], idxs_v.at[pl.ds(0, e_per)])
    pltpu.sync_copy(r_idx.at[pl.ds(base, e_per)], idxr_v.at[pl.ds(0, e_per)])
    _fused_gather_add(ta, tb, idxs_v, idxr_v, out, base, e_per,
                      a0, a1, b0, b1, w0, w1, gsem, wsem)


def _sc_gather_add(ta, tb, s_idx, r_idx):
    ne = s_idx.shape[0]
    e_per = ne // NW
    k = pl.kernel(
        _sc_body,
        out_type=jax.ShapeDtypeStruct((ne, D), jnp.float32),
        mesh=plsc.VectorSubcoreMesh(core_axis_name="c", subcore_axis_name="s",
                                    num_cores=NC, num_subcores=NS),
        scratch_types=[
            pltpu.VMEM((e_per,), jnp.int32),
            pltpu.VMEM((e_per,), jnp.int32),
            pltpu.VMEM((C, D), jnp.float32),
            pltpu.VMEM((C, D), jnp.float32),
            pltpu.VMEM((C, D), jnp.float32),
            pltpu.VMEM((C, D), jnp.float32),
            pltpu.VMEM((C, D), jnp.float32),
            pltpu.VMEM((C, D), jnp.float32),
            pltpu.SemaphoreType.DMA,
            pltpu.SemaphoreType.DMA,
        ],
    )
    return k(ta, tb, s_idx, r_idx)


def _mlp_body(g_ref, e_ref, w1c_ref, w2_ref, b2_ref, out_ref):
    e = e_ref[...]
    h = g_ref[...] + jnp.dot(e, w1c_ref[...], preferred_element_type=jnp.float32)
    h = jnp.maximum(h, 0.0)
    out_ref[...] = e + jnp.dot(h, w2_ref[...],
                               preferred_element_type=jnp.float32) + b2_ref[...]


def _edge_mlp(g, e, w1c, w2, b2, blk):
    n = e.shape[0]
    row_spec = pl.BlockSpec((blk, D), lambda i: (i, 0))
    wspec = pl.BlockSpec((D, D), lambda i: (0, 0))
    bias = pl.BlockSpec((1, D), lambda i: (0, 0))
    return pl.pallas_call(
        _mlp_body,
        grid=(n // blk,),
        in_specs=[row_spec, row_spec, wspec, wspec, bias],
        out_specs=row_spec,
        out_shape=jax.ShapeDtypeStruct((n, D), jnp.float32),
        compiler_params=pltpu.CompilerParams(
            dimension_semantics=("arbitrary",)),
    )(g, e, w1c, w2, b2.reshape(1, D))


def kernel(x, mesh_edge_index, mesh_edge_attr, world_edge_index, world_edge_attr,
           Wm1, bm1, Wm2, bm2, Ww1, bw1, Ww2, bw2):
    pam, pbm, paw, pbw = _precompute_tables(x, Wm1, bm1, Ww1, bw1)
    gm = _sc_gather_add(pam, pbm, mesh_edge_index[0], mesh_edge_index[1])
    gw = _sc_gather_add(paw, pbw, world_edge_index[0], world_edge_index[1])
    mesh_out = _edge_mlp(gm, mesh_edge_attr, Wm1[2 * D:], Wm2, bm2, 4000)
    world_out = _edge_mlp(gw, world_edge_attr, Ww1[2 * D:], Ww2, bw2, 4000)
    return (mesh_out, world_out)


# speedup vs baseline: 1.1604x; 1.1604x over previous
import functools

import jax
import jax.numpy as jnp
from jax import lax
from jax.experimental import pallas as pl
from jax.experimental.pallas import tpu as pltpu
from jax.experimental.pallas import tpu_sc as plsc

D = 128
DW = D // 2
NC, NS = 2, 16
NW = NC * NS
G_SUBS = (128, 72)
G_OFF = (0, 128)
CH = sum(G_SUBS)
NSLOT = 4


def _prep_body(x_ref, wm1_ref, bm1_ref, ww1_ref, bw1_ref,
               pam_ref, pbm_ref, paw_ref, pbw_ref):
    x = x_ref[...]
    f32 = jnp.float32
    pam_ref[...] = jnp.dot(x, wm1_ref[0:D, :], preferred_element_type=f32) + bm1_ref[...]
    pbm_ref[...] = jnp.dot(x, wm1_ref[D:2 * D, :], preferred_element_type=f32)
    paw_ref[...] = jnp.dot(x, ww1_ref[0:D, :], preferred_element_type=f32) + bw1_ref[...]
    pbw_ref[...] = jnp.dot(x, ww1_ref[D:2 * D, :], preferred_element_type=f32)


def _precompute_tables(x, wm1, bm1, ww1, bw1):
    n = x.shape[0]
    blk = n // 5
    tbl = jax.ShapeDtypeStruct((n, D), jnp.float32)
    row_spec = pl.BlockSpec((blk, D), lambda i: (i, 0))
    full = pl.BlockSpec((2 * D, D), lambda i: (0, 0))
    bias = pl.BlockSpec((1, D), lambda i: (0, 0))
    return pl.pallas_call(
        _prep_body,
        grid=(5,),
        in_specs=[row_spec, full, bias, full, bias],
        out_specs=(row_spec, row_spec, row_spec, row_spec),
        out_shape=(tbl, tbl, tbl, tbl),
    )(x, wm1[: 2 * D], bm1.reshape(1, D), ww1[: 2 * D], bw1.reshape(1, D))


def _gather_stream(table, idx_hbm, out_hbm, idx_v, slots, gsem, wsem, base, n_edges):
    n = n_edges // CH
    pltpu.sync_copy(idx_hbm.at[pl.ds(base, n_edges)], idx_v.at[pl.ds(0, n_edges)])

    def issue(jj, slot):
        for off, sub in zip(G_OFF, G_SUBS):
            pltpu.async_copy(
                table.at[idx_v.at[pl.ds(jj * CH + off, sub)]],
                slot.at[pl.ds(off, sub)],
                gsem,
            )

    def wait_gathers(slot):
        for off, sub in zip(G_OFF, G_SUBS):
            pltpu.make_async_copy(table.at[pl.ds(0, sub)],
                                  slot.at[pl.ds(off, sub)], gsem).wait()

    def wait_write(slot):
        pltpu.make_async_copy(slot, out_hbm.at[pl.ds(base, CH)], wsem).wait()

    issue(0, slots[0])

    def step(i, carry):
        for p in range(NSLOT):
            @pl.when(i % NSLOT == p)
            def _():
                wait_gathers(slots[p])

                @pl.when(i >= NSLOT - 1)
                def _():
                    wait_write(slots[(p + 1) % NSLOT])

                @pl.when(i + 1 < n)
                def _():
                    issue(i + 1, slots[(p + 1) % NSLOT])
                pltpu.async_copy(slots[p], out_hbm.at[pl.ds(base + i * CH, CH)], wsem)
        return carry

    lax.fori_loop(0, n, step, 0)
    for _ in range(NSLOT - 1):
        wait_write(slots[0])


def _sc_body(tam, tbm, taw, tbw, sm, rm, sw, rw,
             gam, gbm, gaw, gbw, idx_v, s0, s1, s2, s3, gsem, wsem):
    wid = lax.axis_index("s") * NC + lax.axis_index("c")
    em = sm.shape[0] // NW
    ew = sw.shape[0] // NW
    slots = (s0, s1, s2, s3)
    _gather_stream(tam, sm, gam, idx_v, slots, gsem, wsem, wid * em, em)
    _gather_stream(tbm, rm, gbm, idx_v, slots, gsem, wsem, wid * em, em)
    _gather_stream(taw, sw, gaw, idx_v, slots, gsem, wsem, wid * ew, ew)
    _gather_stream(tbw, rw, gbw, idx_v, slots, gsem, wsem, wid * ew, ew)


def _sc_gather(tam, tbm, taw, tbw, sm, rm, sw, rw):
    em, ew = sm.shape[0], sw.shape[0]
    out = (jax.ShapeDtypeStruct((em, D), jnp.float32),
           jax.ShapeDtypeStruct((em, D), jnp.float32),
           jax.ShapeDtypeStruct((ew, D), jnp.float32),
           jax.ShapeDtypeStruct((ew, D), jnp.float32))
    k = pl.kernel(
        _sc_body,
        out_type=out,
        mesh=plsc.VectorSubcoreMesh(core_axis_name="c", subcore_axis_name="s",
                                    num_cores=NC, num_subcores=NS),
        scratch_types=[
            pltpu.VMEM((em // NW,), jnp.int32),
            pltpu.VMEM((CH, D), jnp.float32),
            pltpu.VMEM((CH, D), jnp.float32),
            pltpu.VMEM((CH, D), jnp.float32),
            pltpu.VMEM((CH, D), jnp.float32),
            pltpu.SemaphoreType.DMA,
            pltpu.SemaphoreType.DMA,
        ],
    )
    return k(tam, tbm, taw, tbw, sm, rm, sw, rw)


def _mlp_body(ga_ref, gb_ref, e_ref, w1c_ref, w2_ref, b2_ref, out_ref):
    e = e_ref[...]
    bf = jnp.bfloat16
    h = (ga_ref[...].astype(jnp.float32) + gb_ref[...].astype(jnp.float32)
         + jnp.dot(e.astype(bf), w1c_ref[...].astype(bf),
                   preferred_element_type=jnp.float32))
    h = jnp.maximum(h, 0.0)
    out_ref[...] = e + jnp.dot(h.astype(bf), w2_ref[...].astype(bf),
                               preferred_element_type=jnp.float32) + b2_ref[...]


def _edge_mlp(ga, gb, e, w1c, w2, b2, blk):
    n = e.shape[0]
    gspec = pl.BlockSpec((blk, D), lambda i: (i, 0))
    wspec = pl.BlockSpec((D, D), lambda i: (0, 0))
    bias = pl.BlockSpec((1, D), lambda i: (0, 0))
    return pl.pallas_call(
        _mlp_body,
        grid=(n // blk,),
        in_specs=[gspec, gspec, gspec, wspec, wspec, bias],
        out_specs=gspec,
        out_shape=jax.ShapeDtypeStruct((n, D), jnp.float32),
        compiler_params=pltpu.CompilerParams(
            dimension_semantics=("arbitrary",)),
    )(ga, gb, e, w1c, w2, b2.reshape(1, D))


def kernel(x, mesh_edge_index, mesh_edge_attr, world_edge_index, world_edge_attr,
           Wm1, bm1, Wm2, bm2, Ww1, bw1, Ww2, bw2):
    pam, pbm, paw, pbw = _precompute_tables(x, Wm1, bm1, Ww1, bw1)
    gam, gbm, gaw, gbw = _sc_gather(
        pam, pbm, paw, pbw,
        mesh_edge_index[0], mesh_edge_index[1],
        world_edge_index[0], world_edge_index[1])
    mesh_out = _edge_mlp(gam, gbm, mesh_edge_attr, Wm1[2 * D:], Wm2, bm2, 4000)
    world_out = _edge_mlp(gaw, gbw, world_edge_attr, Ww1[2 * D:], Ww2, bw2, 4000)
    return (mesh_out, world_out)


# speedup vs baseline: 1.1896x; 1.0251x over previous
import functools

import jax
import jax.numpy as jnp
from jax import lax
from jax.experimental import pallas as pl
from jax.experimental.pallas import tpu as pltpu
from jax.experimental.pallas import tpu_sc as plsc

D = 128
DW = D // 2
NC, NS = 2, 16
NW = NC * NS
G_SUBS = (128, 72)
G_OFF = (0, 128)
CH = sum(G_SUBS)
NSLOT = 4


def _prep_body(x_ref, wm1_ref, bm1_ref, ww1_ref, bw1_ref,
               pam_ref, pbm_ref, paw_ref, pbw_ref):
    x = x_ref[...]
    f32 = jnp.float32
    pam_ref[...] = jnp.dot(x, wm1_ref[0:D, :], preferred_element_type=f32) + bm1_ref[...]
    pbm_ref[...] = jnp.dot(x, wm1_ref[D:2 * D, :], preferred_element_type=f32)
    paw_ref[...] = jnp.dot(x, ww1_ref[0:D, :], preferred_element_type=f32) + bw1_ref[...]
    pbw_ref[...] = jnp.dot(x, ww1_ref[D:2 * D, :], preferred_element_type=f32)


def _precompute_tables(x, wm1, bm1, ww1, bw1):
    n = x.shape[0]
    blk = n // 5
    tbl = jax.ShapeDtypeStruct((n, D), jnp.float32)
    row_spec = pl.BlockSpec((blk, D), lambda i: (i, 0))
    full = pl.BlockSpec((2 * D, D), lambda i: (0, 0))
    bias = pl.BlockSpec((1, D), lambda i: (0, 0))
    return pl.pallas_call(
        _prep_body,
        grid=(5,),
        in_specs=[row_spec, full, bias, full, bias],
        out_specs=(row_spec, row_spec, row_spec, row_spec),
        out_shape=(tbl, tbl, tbl, tbl),
    )(x, wm1[: 2 * D], bm1.reshape(1, D), ww1[: 2 * D], bw1.reshape(1, D))


def _gather_stream(table, idx_hbm, out_hbm, idx_v, slots, gsem, wsem, base, n_edges):
    n = n_edges // CH
    pltpu.sync_copy(idx_hbm.at[pl.ds(base, n_edges)], idx_v.at[pl.ds(0, n_edges)])

    def issue(jj, slot):
        for off, sub in zip(G_OFF, G_SUBS):
            pltpu.async_copy(
                table.at[idx_v.at[pl.ds(jj * CH + off, sub)]],
                slot.at[pl.ds(off, sub)],
                gsem,
            )

    def wait_gathers(slot):
        for off, sub in zip(G_OFF, G_SUBS):
            pltpu.make_async_copy(table.at[pl.ds(0, sub)],
                                  slot.at[pl.ds(off, sub)], gsem).wait()

    def wait_write(slot):
        pltpu.make_async_copy(slot, out_hbm.at[pl.ds(base, CH)], wsem).wait()

    issue(0, slots[0])

    def step(i, carry):
        for p in range(NSLOT):
            @pl.when(i % NSLOT == p)
            def _():
                wait_gathers(slots[p])

                @pl.when(i >= NSLOT - 1)
                def _():
                    wait_write(slots[(p + 1) % NSLOT])

                @pl.when(i + 1 < n)
                def _():
                    issue(i + 1, slots[(p + 1) % NSLOT])
                pltpu.async_copy(slots[p], out_hbm.at[pl.ds(base + i * CH, CH)], wsem)
        return carry

    lax.fori_loop(0, n, step, 0)
    for _ in range(NSLOT - 1):
        wait_write(slots[0])


def _sc_body(ta, tb, s_idx, r_idx, ga, gb, idx_v, s0, s1, s2, s3, gsem, wsem):
    wid = lax.axis_index("s") * NC + lax.axis_index("c")
    ne = s_idx.shape[0] // NW
    slots = (s0, s1, s2, s3)
    _gather_stream(ta, s_idx, ga, idx_v, slots, gsem, wsem, wid * ne, ne)
    _gather_stream(tb, r_idx, gb, idx_v, slots, gsem, wsem, wid * ne, ne)


def _sc_gather(ta, tb, eidx):
    ne = eidx.shape[1]
    out = (jax.ShapeDtypeStruct((ne, D), jnp.float32),
           jax.ShapeDtypeStruct((ne, D), jnp.float32))
    k = pl.kernel(
        _sc_body,
        out_type=out,
        mesh=plsc.VectorSubcoreMesh(core_axis_name="c", subcore_axis_name="s",
                                    num_cores=NC, num_subcores=NS),
        scratch_types=[
            pltpu.VMEM((ne // NW,), jnp.int32),
            pltpu.VMEM((CH, D), jnp.float32),
            pltpu.VMEM((CH, D), jnp.float32),
            pltpu.VMEM((CH, D), jnp.float32),
            pltpu.VMEM((CH, D), jnp.float32),
            pltpu.SemaphoreType.DMA,
            pltpu.SemaphoreType.DMA,
        ],
    )
    return k(ta, tb, eidx[0], eidx[1])


def _mlp_body(ga_ref, gb_ref, e_ref, w1c_ref, w2_ref, b2_ref, out_ref):
    e = e_ref[...]
    bf = jnp.bfloat16
    h = (ga_ref[...].astype(jnp.float32) + gb_ref[...].astype(jnp.float32)
         + jnp.dot(e.astype(bf), w1c_ref[...].astype(bf),
                   preferred_element_type=jnp.float32))
    h = jnp.maximum(h, 0.0)
    out_ref[...] = e + jnp.dot(h.astype(bf), w2_ref[...].astype(bf),
                               preferred_element_type=jnp.float32) + b2_ref[...]


def _edge_mlp(ga, gb, e, w1c, w2, b2, blk):
    n = e.shape[0]
    gspec = pl.BlockSpec((blk, D), lambda i: (i, 0))
    wspec = pl.BlockSpec((D, D), lambda i: (0, 0))
    bias = pl.BlockSpec((1, D), lambda i: (0, 0))
    return pl.pallas_call(
        _mlp_body,
        grid=(n // blk,),
        in_specs=[gspec, gspec, gspec, wspec, wspec, bias],
        out_specs=gspec,
        out_shape=jax.ShapeDtypeStruct((n, D), jnp.float32),
        compiler_params=pltpu.CompilerParams(
            dimension_semantics=("arbitrary",)),
    )(ga, gb, e, w1c, w2, b2.reshape(1, D))


def kernel(x, mesh_edge_index, mesh_edge_attr, world_edge_index, world_edge_attr,
           Wm1, bm1, Wm2, bm2, Ww1, bw1, Ww2, bw2):
    pam, pbm, paw, pbw = _precompute_tables(x, Wm1, bm1, Ww1, bw1)
    gam, gbm = _sc_gather(pam, pbm, mesh_edge_index)
    gaw, gbw = _sc_gather(paw, pbw, world_edge_index)
    mesh_out = _edge_mlp(gam, gbm, mesh_edge_attr, Wm1[2 * D:], Wm2, bm2, 8000)
    world_out = _edge_mlp(gaw, gbw, world_edge_attr, Ww1[2 * D:], Ww2, bw2, 8000)
    return (mesh_out, world_out)


# speedup vs baseline: 1.2186x; 1.0244x over previous
import functools

import jax
import jax.numpy as jnp
from jax import lax
from jax.experimental import pallas as pl
from jax.experimental.pallas import tpu as pltpu
from jax.experimental.pallas import tpu_sc as plsc

D = 128
DW = D // 2
NC, NS = 2, 16
NW = NC * NS
G_SUBS = (128, 72)
G_OFF = (0, 128)
CH = sum(G_SUBS)
NSLOT = 4


def _prep_body(x_ref, wm1_ref, bm1_ref, ww1_ref, bw1_ref,
               pam_ref, pbm_ref, paw_ref, pbw_ref):
    x = x_ref[...]
    f32 = jnp.float32
    pam_ref[...] = jnp.dot(x, wm1_ref[0:D, :], preferred_element_type=f32) + bm1_ref[...]
    pbm_ref[...] = jnp.dot(x, wm1_ref[D:2 * D, :], preferred_element_type=f32)
    paw_ref[...] = jnp.dot(x, ww1_ref[0:D, :], preferred_element_type=f32) + bw1_ref[...]
    pbw_ref[...] = jnp.dot(x, ww1_ref[D:2 * D, :], preferred_element_type=f32)


def _precompute_tables(x, wm1, bm1, ww1, bw1):
    n = x.shape[0]
    blk = n // 5
    tbl = jax.ShapeDtypeStruct((n, D), jnp.float32)
    row_spec = pl.BlockSpec((blk, D), lambda i: (i, 0))
    full = pl.BlockSpec((2 * D, D), lambda i: (0, 0))
    bias = pl.BlockSpec((1, D), lambda i: (0, 0))
    return pl.pallas_call(
        _prep_body,
        grid=(5,),
        in_specs=[row_spec, full, bias, full, bias],
        out_specs=(row_spec, row_spec, row_spec, row_spec),
        out_shape=(tbl, tbl, tbl, tbl),
    )(x, wm1[: 2 * D], bm1.reshape(1, D), ww1[: 2 * D], bw1.reshape(1, D))


def _gather_stream(table, idx_hbm, out_hbm, idx_v, slots, gsem, wsem, base, n_edges):
    n = n_edges // CH
    pltpu.sync_copy(idx_hbm.at[pl.ds(base, n_edges)], idx_v.at[pl.ds(0, n_edges)])

    def issue(jj, slot):
        for off, sub in zip(G_OFF, G_SUBS):
            pltpu.async_copy(
                table.at[idx_v.at[pl.ds(jj * CH + off, sub)]],
                slot.at[pl.ds(off, sub)],
                gsem,
            )

    def wait_gathers(slot):
        for off, sub in zip(G_OFF, G_SUBS):
            pltpu.make_async_copy(table.at[pl.ds(0, sub)],
                                  slot.at[pl.ds(off, sub)], gsem).wait()

    def wait_write(slot):
        pltpu.make_async_copy(slot, out_hbm.at[pl.ds(base, CH)], wsem).wait()

    issue(0, slots[0])
    issue(1, slots[1])

    def step(i, carry):
        for p in range(NSLOT):
            @pl.when(i % NSLOT == p)
            def _():
                wait_gathers(slots[p])

                @pl.when(i >= 2)
                def _():
                    wait_write(slots[(p + 2) % NSLOT])

                @pl.when(i + 2 < n)
                def _():
                    issue(i + 2, slots[(p + 2) % NSLOT])
                pltpu.async_copy(slots[p], out_hbm.at[pl.ds(base + i * CH, CH)], wsem)
        return carry

    lax.fori_loop(0, n, step, 0)
    wait_write(slots[0])
    wait_write(slots[0])


def _sc_body(ta, tb, s_idx, r_idx, ga, gb, idx_v, s0, s1, s2, s3, gsem, wsem):
    wid = lax.axis_index("s") * NC + lax.axis_index("c")
    ne = s_idx.shape[0] // NW
    slots = (s0, s1, s2, s3)
    _gather_stream(ta, s_idx, ga, idx_v, slots, gsem, wsem, wid * ne, ne)
    _gather_stream(tb, r_idx, gb, idx_v, slots, gsem, wsem, wid * ne, ne)


def _sc_gather(ta, tb, eidx):
    ne = eidx.shape[1]
    out = (jax.ShapeDtypeStruct((ne, D), jnp.float32),
           jax.ShapeDtypeStruct((ne, D), jnp.float32))
    k = pl.kernel(
        _sc_body,
        out_type=out,
        mesh=plsc.VectorSubcoreMesh(core_axis_name="c", subcore_axis_name="s",
                                    num_cores=NC, num_subcores=NS),
        scratch_types=[
            pltpu.VMEM((ne // NW,), jnp.int32),
            pltpu.VMEM((CH, D), jnp.float32),
            pltpu.VMEM((CH, D), jnp.float32),
            pltpu.VMEM((CH, D), jnp.float32),
            pltpu.VMEM((CH, D), jnp.float32),
            pltpu.SemaphoreType.DMA,
            pltpu.SemaphoreType.DMA,
        ],
    )
    return k(ta, tb, eidx[0], eidx[1])


def _mlp_body(ga_ref, gb_ref, e_ref, w1c_ref, w2_ref, b2_ref, out_ref):
    e = e_ref[...]
    bf = jnp.bfloat16
    h = (ga_ref[...].astype(jnp.float32) + gb_ref[...].astype(jnp.float32)
         + jnp.dot(e.astype(bf), w1c_ref[...].astype(bf),
                   preferred_element_type=jnp.float32))
    h = jnp.maximum(h, 0.0)
    out_ref[...] = e + jnp.dot(h.astype(bf), w2_ref[...].astype(bf),
                               preferred_element_type=jnp.float32) + b2_ref[...]


def _edge_mlp(ga, gb, e, w1c, w2, b2, blk):
    n = e.shape[0]
    gspec = pl.BlockSpec((blk, D), lambda i: (i, 0))
    wspec = pl.BlockSpec((D, D), lambda i: (0, 0))
    bias = pl.BlockSpec((1, D), lambda i: (0, 0))
    return pl.pallas_call(
        _mlp_body,
        grid=(n // blk,),
        in_specs=[gspec, gspec, gspec, wspec, wspec, bias],
        out_specs=gspec,
        out_shape=jax.ShapeDtypeStruct((n, D), jnp.float32),
        compiler_params=pltpu.CompilerParams(
            dimension_semantics=("arbitrary",)),
    )(ga, gb, e, w1c, w2, b2.reshape(1, D))


def kernel(x, mesh_edge_index, mesh_edge_attr, world_edge_index, world_edge_attr,
           Wm1, bm1, Wm2, bm2, Ww1, bw1, Ww2, bw2):
    pam, pbm, paw, pbw = _precompute_tables(x, Wm1, bm1, Ww1, bw1)
    gam, gbm = _sc_gather(pam, pbm, mesh_edge_index)
    gaw, gbw = _sc_gather(paw, pbw, world_edge_index)
    mesh_out = _edge_mlp(gam, gbm, mesh_edge_attr, Wm1[2 * D:], Wm2, bm2, 4000)
    world_out = _edge_mlp(gaw, gbw, world_edge_attr, Ww1[2 * D:], Ww2, bw2, 4000)
    return (mesh_out, world_out)
